# baseline (speedup 1.0000x reference)
import functools

import jax
import jax.numpy as jnp
from jax import lax
from jax.experimental import pallas as pl
from jax.experimental.pallas import tpu as pltpu
from jax.experimental.pallas import tpu_sc as plsc

N = 262144
FEATURE_DIM = 64
TOKEN_DIM = 64
HIDDEN = 64
MAX_TOKENS = 1024
TILE = 4096


def _importance_body(feat_ref, c4_ref, mw1_ref, mb1_ref, mw2_ref, mb2_ref,
                     iw1f_ref, iw1c_ref, ib1_ref, g_ref, b_ref,
                     iw2_ref, ib2_ref, iw3_ref, ib3_ref, imp_ref):
    f = feat_ref[...]
    pf = jnp.maximum(jnp.dot(f, mw1_ref[...], preferred_element_type=jnp.float32) + mb1_ref[...], 0.0)
    pf = jnp.dot(pf, mw2_ref[...], preferred_element_type=jnp.float32) + mb2_ref[...]
    c4 = c4_ref[...]
    h = (jnp.dot(pf, iw1f_ref[...], preferred_element_type=jnp.float32)
         + jnp.dot(c4, iw1c_ref[...], preferred_element_type=jnp.float32)
         + ib1_ref[...])
    h = jnp.maximum(h, 0.0)
    mu = jnp.mean(h, axis=-1, keepdims=True)
    var = jnp.mean((h - mu) ** 2, axis=-1, keepdims=True)
    h = (h - mu) / jnp.sqrt(var + 1e-5) * g_ref[...] + b_ref[...]
    h = jnp.maximum(jnp.dot(h, iw2_ref[...], preferred_element_type=jnp.float32) + ib2_ref[...], 0.0)
    imp = jnp.dot(h, iw3_ref[...], preferred_element_type=jnp.float32) + ib3_ref[...]
    b = jax.lax.bitcast_convert_type(imp[:, 0], jnp.uint32)
    neg = b >= jnp.uint32(0x80000000)
    imp_ref[...] = jnp.where(neg, b, (~b) & jnp.uint32(0x7FFFFFFF))


def _importance(features, coords4, mlp_W1, mlp_b1, mlp_W2, mlp_b2,
                imp_W1f, imp_W1c, imp_b1, ln_g, ln_b, imp_W2, imp_b2,
                imp_W3, imp_b3):
    grid = N // TILE
    wspec = lambda shape: pl.BlockSpec(shape, lambda i: tuple(0 for _ in shape))
    return pl.pallas_call(
        _importance_body,
        grid=(grid,),
        in_specs=[
            pl.BlockSpec((TILE, FEATURE_DIM), lambda i: (i, 0)),
            pl.BlockSpec((TILE, 4), lambda i: (i, 0)),
            wspec((FEATURE_DIM, HIDDEN)), wspec((HIDDEN,)),
            wspec((HIDDEN, TOKEN_DIM)), wspec((TOKEN_DIM,)),
            wspec((TOKEN_DIM, HIDDEN)), wspec((4, HIDDEN)), wspec((HIDDEN,)),
            wspec((HIDDEN,)), wspec((HIDDEN,)),
            wspec((HIDDEN, HIDDEN)), wspec((HIDDEN,)),
            wspec((HIDDEN, 1)), wspec((1,)),
        ],
        out_specs=pl.BlockSpec((TILE,), lambda i: (i,)),
        out_shape=jax.ShapeDtypeStruct((N,), jnp.uint32),
        compiler_params=pltpu.CompilerParams(
            dimension_semantics=("arbitrary",),
        ),
    )(features, coords4, mlp_W1, mlp_b1, mlp_W2, mlp_b2,
      imp_W1f, imp_W1c, imp_b1, ln_g, ln_b, imp_W2, imp_b2, imp_W3, imp_b3)


NSUB = 16
CHUNK = N // NSUB
NVREG = CHUNK // 16
OUTPAD = 1040
DUMP = OUTPAD - 1


def _lane_sum(v, pad_ref):
    r = v
    for s in (8, 4, 2, 1):
        pad_ref[pl.ds(0, 16)] = r
        r = r + pad_ref[pl.ds(s, 16)]
    return r[0]


def _lane_min_u32(v, pad_ref):
    r = v
    for s in (8, 4, 2, 1):
        pad_ref[pl.ds(0, 16)] = r
        r = jnp.minimum(r, pad_ref[pl.ds(s, 16)])
    return r[0]


def _sc_topk(keys):
    mesh = plsc.VectorSubcoreMesh(core_axis_name="c", subcore_axis_name="s",
                                  num_cores=1)

    @functools.partial(
        pl.kernel, mesh=mesh,
        out_type=jax.ShapeDtypeStruct((MAX_TOKENS,), jnp.int32),
        scratch_types=[
            pltpu.VMEM((CHUNK,), jnp.uint32),
            pltpu.VMEM((32,), jnp.int32),
            pltpu.VMEM((32,), jnp.uint32),
            pltpu.VMEM((16,), jnp.int32),
            pltpu.VMEM((NSUB, 16), jnp.int32),
            pltpu.VMEM((OUTPAD,), jnp.int32),
            pltpu.VMEM((OUTPAD + 8,), jnp.int32),
            pltpu.VMEM((MAX_TOKENS + 32,), jnp.int32),
            pltpu.VMEM_SHARED((NSUB, 16), jnp.int32),
            pltpu.VMEM_SHARED((2208,), jnp.int32),
        ],
    )
    def topk(key_hbm, sel_hbm, key_v, padi_v, padu_v, pub_v, cnts_v,
             outb_v, stage_v, outf_v, cnts_sh, sel_sh):
        tid = lax.axis_index("s")
        lanes = lax.iota(jnp.int32, 16)
        base = tid * CHUNK

        pltpu.sync_copy(key_hbm.at[pl.ds(base, CHUNK)], key_v)
        padi_v[pl.ds(16, 16)] = jnp.zeros((16,), jnp.int32)
        padu_v[pl.ds(16, 16)] = jnp.full((16,), 0xFFFFFFFF, jnp.uint32)

        def count_le(mid):
            def outer(i, tot):
                acc = jnp.zeros((16,), jnp.int32)
                for j in range(16):
                    k = key_v[pl.ds(i * 256 + j * 16, 16)]
                    acc = acc + jnp.where(k <= mid, 1, 0)
                return tot + _lane_sum(acc, padi_v)
            return lax.fori_loop(0, NVREG // 16, outer, jnp.int32(0))

        def publish_and_sum(val0, val1):
            pub_v[...] = jnp.where(lanes == 0, val0,
                                   jnp.where(lanes == 1, val1, 0))
            pltpu.sync_copy(pub_v, cnts_sh.at[tid])
            plsc.subcore_barrier()
            pltpu.sync_copy(cnts_sh, cnts_v)
            plsc.subcore_barrier()

        def round_(r, carry):
            lo, hi = carry
            mid = lo + lax.shift_right_logical(hi - lo, jnp.uint32(1))
            c_loc = count_le(mid)
            publish_and_sum(c_loc, jnp.int32(0))
            c_glob = jnp.int32(0)
            for j in range(NSUB):
                c_glob = c_glob + cnts_v[j, :][0]
            ge = c_glob >= jnp.int32(MAX_TOKENS)
            lo2 = jnp.where(ge, lo, mid + jnp.uint32(1))
            hi2 = jnp.where(ge, mid, hi)
            return lo2, hi2
        thr, _ = lax.fori_loop(0, 32, round_,
                               (jnp.uint32(0), jnp.uint32(0xFFFFFFFF)))

        def cnt_outer(i, carry):
            ts, tt = carry
            accs = jnp.zeros((16,), jnp.int32)
            acct = jnp.zeros((16,), jnp.int32)
            for j in range(16):
                k = key_v[pl.ds(i * 256 + j * 16, 16)]
                accs = accs + jnp.where(k < thr, 1, 0)
                acct = acct + jnp.where(k == thr, 1, 0)
            return (ts + _lane_sum(accs, padi_v), tt + _lane_sum(acct, padi_v))
        s_i, t_i = lax.fori_loop(0, NVREG // 16, cnt_outer,
                                 (jnp.int32(0), jnp.int32(0)))
        publish_and_sum(s_i, t_i)

        S_tot = jnp.int32(0)
        Tp_i = jnp.int32(0)
        for j in range(NSUB):
            row = cnts_v[j, :]
            S_tot = S_tot + row[0]
            Tp_i = Tp_i + jnp.where(jnp.int32(j) < tid, row[1], 0)
        n_eq_take = jnp.int32(MAX_TOKENS) - S_tot
        quota_i = jnp.clip(n_eq_take - Tp_i, 0, t_i)
        Op_i = jnp.int32(0)
        Tp_run = jnp.int32(0)
        for j in range(NSUB):
            row = cnts_v[j, :]
            cj = row[0] + jnp.clip(n_eq_take - Tp_run, 0, row[1])
            cpj = ((cj + 7) >> 3) << 3
            Op_i = Op_i + jnp.where(jnp.int32(j) < tid, cpj, 0)
            Tp_run = Tp_run + row[1]

        def comp(i, carry):
            def active(carry):
                p, nt = carry
                k = key_v[pl.ds(i * 16, 16)]
                for l in range(16):
                    kv = k[l]
                    is_tie = kv == thr
                    sel = jnp.logical_or(
                        kv < thr, jnp.logical_and(is_tie, nt < quota_i))
                    p_eff = jnp.where(sel, p, jnp.int32(DUMP))
                    b16 = p_eff & ~jnp.int32(15)
                    ln = p_eff & jnp.int32(15)
                    old = outb_v[pl.ds(b16, 16)]
                    outb_v[pl.ds(b16, 16)] = jnp.where(
                        lanes == ln, base + i * 16 + l, old)
                    p = p + jnp.where(sel, 1, 0)
                    nt = nt + jnp.where(is_tie, 1, 0)
                return p, nt
            k = key_v[pl.ds(i * 16, 16)]
            kmin = _lane_min_u32(k, padu_v)
            return lax.cond(kmin <= thr, active, lambda c: c, carry)
        lax.fori_loop(0, NVREG, comp, (jnp.int32(0), jnp.int32(0)))

        for t in range(NSUB):
            @pl.when(tid == jnp.int32(t))
            def _():
                pltpu.sync_copy(
                    outb_v, sel_sh.at[pl.ds(pl.multiple_of(Op_i, 8), OUTPAD)])
            plsc.subcore_barrier()

        @pl.when(tid == 0)
        def _():
            Oe = jnp.int32(0)
            Op = jnp.int32(0)
            Tp2 = jnp.int32(0)
            for j in range(NSUB):
                row = cnts_v[j, :]
                cj = row[0] + jnp.clip(n_eq_take - Tp2, 0, row[1])
                pltpu.sync_copy(
                    sel_sh.at[pl.ds(pl.multiple_of(Op, 8), OUTPAD + 8)],
                    stage_v)
                def mv(q, _, cj=cj, Oe=Oe):
                    outf_v[pl.ds(Oe + q * 16, 16)] = stage_v[pl.ds(q * 16, 16)]
                    return 0
                lax.fori_loop(0, (cj + 15) >> 4, mv, 0)
                Oe = Oe + cj
                Op = Op + (((cj + 7) >> 3) << 3)
                Tp2 = Tp2 + row[1]
            pltpu.sync_copy(outf_v.at[pl.ds(0, MAX_TOKENS)], sel_hbm)

    return topk(keys)


def kernel(coordinates, features, temperature, mlp_W1, mlp_b1, mlp_W2, mlp_b2,
           imp_W1, imp_b1, ln_g, ln_b, imp_W2, imp_b2, imp_W3, imp_b3,
           nb_W1, nb_b1, nb_W2, nb_b2):
    coords4 = coordinates[:, 1:5]
    importance = _importance(
        features, coords4, mlp_W1, mlp_b1, mlp_W2, mlp_b2,
        imp_W1[:TOKEN_DIM], imp_W1[TOKEN_DIM:], imp_b1, ln_g, ln_b,
        imp_W2, imp_b2, imp_W3, imp_b3)

    sel = _sc_topk(importance)
    cents = coords4[sel]
    fsel = features[sel]
    pf_sel = jnp.maximum(fsel @ mlp_W1 + mlp_b1, 0.0) @ mlp_W2 + mlp_b2
    toks = jnp.maximum(pf_sel @ nb_W1 + nb_b1, 0.0) @ nb_W2 + nb_b2
    order = jnp.argsort(cents[:, 3])
    cents = cents[order]
    toks = toks[order]
    tokens = toks[None]
    centroids = cents[None]
    masks = jnp.ones((1, MAX_TOKENS), dtype=bool)
    return tokens, centroids, masks


# speedup vs baseline: 1.0151x; 1.0151x over previous
import functools

import jax
import jax.numpy as jnp
from jax import lax
from jax.experimental import pallas as pl
from jax.experimental.pallas import tpu as pltpu
from jax.experimental.pallas import tpu_sc as plsc

N = 262144
FEATURE_DIM = 64
TOKEN_DIM = 64
HIDDEN = 64
MAX_TOKENS = 1024
TILE = 8192


def _importance_body(feat_ref, c4_ref, mw1_ref, mb1_ref, mw2_ref, mb2_ref,
                     iw1f_ref, iw1c_ref, ib1_ref, g_ref, b_ref,
                     iw2_ref, ib2_ref, iw3_ref, ib3_ref, imp_ref):
    f = feat_ref[...]
    pf = jnp.maximum(jnp.dot(f, mw1_ref[...], preferred_element_type=jnp.float32) + mb1_ref[...], 0.0)
    pf = jnp.dot(pf, mw2_ref[...], preferred_element_type=jnp.float32) + mb2_ref[...]
    c4 = c4_ref[...]
    h = (jnp.dot(pf, iw1f_ref[...], preferred_element_type=jnp.float32)
         + jnp.dot(c4, iw1c_ref[...], preferred_element_type=jnp.float32)
         + ib1_ref[...])
    h = jnp.maximum(h, 0.0)
    mu = jnp.mean(h, axis=-1, keepdims=True)
    var = jnp.mean((h - mu) ** 2, axis=-1, keepdims=True)
    h = (h - mu) / jnp.sqrt(var + 1e-5) * g_ref[...] + b_ref[...]
    h = jnp.maximum(jnp.dot(h, iw2_ref[...], preferred_element_type=jnp.float32) + ib2_ref[...], 0.0)
    imp = jnp.dot(h, iw3_ref[...], preferred_element_type=jnp.float32) + ib3_ref[...]
    b = jax.lax.bitcast_convert_type(imp[:, 0], jnp.uint32)
    neg = b >= jnp.uint32(0x80000000)
    imp_ref[...] = jnp.where(neg, b, (~b) & jnp.uint32(0x7FFFFFFF))


def _importance(features, coords4, mlp_W1, mlp_b1, mlp_W2, mlp_b2,
                imp_W1f, imp_W1c, imp_b1, ln_g, ln_b, imp_W2, imp_b2,
                imp_W3, imp_b3):
    grid = N // TILE
    wspec = lambda shape: pl.BlockSpec(shape, lambda i: tuple(0 for _ in shape))
    return pl.pallas_call(
        _importance_body,
        grid=(grid,),
        in_specs=[
            pl.BlockSpec((TILE, FEATURE_DIM), lambda i: (i, 0)),
            pl.BlockSpec((TILE, 4), lambda i: (i, 0)),
            wspec((FEATURE_DIM, HIDDEN)), wspec((HIDDEN,)),
            wspec((HIDDEN, TOKEN_DIM)), wspec((TOKEN_DIM,)),
            wspec((TOKEN_DIM, HIDDEN)), wspec((4, HIDDEN)), wspec((HIDDEN,)),
            wspec((HIDDEN,)), wspec((HIDDEN,)),
            wspec((HIDDEN, HIDDEN)), wspec((HIDDEN,)),
            wspec((HIDDEN, 1)), wspec((1,)),
        ],
        out_specs=pl.BlockSpec((TILE,), lambda i: (i,)),
        out_shape=jax.ShapeDtypeStruct((N,), jnp.uint32),
        compiler_params=pltpu.CompilerParams(
            dimension_semantics=("arbitrary",),
        ),
    )(features, coords4, mlp_W1, mlp_b1, mlp_W2, mlp_b2,
      imp_W1f, imp_W1c, imp_b1, ln_g, ln_b, imp_W2, imp_b2, imp_W3, imp_b3)


NSUB = 16
CHUNK = N // NSUB
NVREG = CHUNK // 16
OUTPAD = 1040
DUMP = OUTPAD - 1


def _lane_sum(v, pad_ref):
    r = v
    for s in (8, 4, 2, 1):
        pad_ref[pl.ds(0, 16)] = r
        r = r + pad_ref[pl.ds(s, 16)]
    return r[0]


def _lane_min_u32(v, pad_ref):
    r = v
    for s in (8, 4, 2, 1):
        pad_ref[pl.ds(0, 16)] = r
        r = jnp.minimum(r, pad_ref[pl.ds(s, 16)])
    return r[0]


def _sc_topk(keys):
    mesh = plsc.VectorSubcoreMesh(core_axis_name="c", subcore_axis_name="s",
                                  num_cores=1)

    @functools.partial(
        pl.kernel, mesh=mesh,
        out_type=jax.ShapeDtypeStruct((MAX_TOKENS,), jnp.int32),
        scratch_types=[
            pltpu.VMEM((CHUNK,), jnp.uint32),
            pltpu.VMEM((32,), jnp.int32),
            pltpu.VMEM((32,), jnp.uint32),
            pltpu.VMEM((16,), jnp.int32),
            pltpu.VMEM((NSUB, 16), jnp.int32),
            pltpu.VMEM((OUTPAD,), jnp.int32),
            pltpu.VMEM((OUTPAD + 8,), jnp.int32),
            pltpu.VMEM((MAX_TOKENS + 32,), jnp.int32),
            pltpu.VMEM_SHARED((NSUB, 16), jnp.int32),
            pltpu.VMEM_SHARED((2208,), jnp.int32),
        ],
    )
    def topk(key_hbm, sel_hbm, key_v, padi_v, padu_v, pub_v, cnts_v,
             outb_v, stage_v, outf_v, cnts_sh, sel_sh):
        tid = lax.axis_index("s")
        lanes = lax.iota(jnp.int32, 16)
        base = tid * CHUNK

        pltpu.sync_copy(key_hbm.at[pl.ds(base, CHUNK)], key_v)
        padi_v[pl.ds(16, 16)] = jnp.zeros((16,), jnp.int32)
        padu_v[pl.ds(16, 16)] = jnp.full((16,), 0xFFFFFFFF, jnp.uint32)

        def count_le(mid):
            def outer(i, tot):
                acc = jnp.zeros((16,), jnp.int32)
                for j in range(16):
                    k = key_v[pl.ds(i * 256 + j * 16, 16)]
                    acc = acc + jnp.where(k <= mid, 1, 0)
                return tot + _lane_sum(acc, padi_v)
            return lax.fori_loop(0, NVREG // 16, outer, jnp.int32(0))

        def publish_and_sum(val0, val1):
            pub_v[...] = jnp.where(lanes == 0, val0,
                                   jnp.where(lanes == 1, val1, 0))
            pltpu.sync_copy(pub_v, cnts_sh.at[tid])
            plsc.subcore_barrier()
            pltpu.sync_copy(cnts_sh, cnts_v)
            plsc.subcore_barrier()

        def round_(r, carry):
            lo, hi = carry
            mid = lo + lax.shift_right_logical(hi - lo, jnp.uint32(1))
            c_loc = count_le(mid)
            publish_and_sum(c_loc, jnp.int32(0))
            c_glob = jnp.int32(0)
            for j in range(NSUB):
                c_glob = c_glob + cnts_v[j, :][0]
            ge = c_glob >= jnp.int32(MAX_TOKENS)
            lo2 = jnp.where(ge, lo, mid + jnp.uint32(1))
            hi2 = jnp.where(ge, mid, hi)
            return lo2, hi2
        thr, _ = lax.fori_loop(0, 32, round_,
                               (jnp.uint32(0), jnp.uint32(0xFFFFFFFF)))

        def cnt_outer(i, carry):
            ts, tt = carry
            accs = jnp.zeros((16,), jnp.int32)
            acct = jnp.zeros((16,), jnp.int32)
            for j in range(16):
                k = key_v[pl.ds(i * 256 + j * 16, 16)]
                accs = accs + jnp.where(k < thr, 1, 0)
                acct = acct + jnp.where(k == thr, 1, 0)
            return (ts + _lane_sum(accs, padi_v), tt + _lane_sum(acct, padi_v))
        s_i, t_i = lax.fori_loop(0, NVREG // 16, cnt_outer,
                                 (jnp.int32(0), jnp.int32(0)))
        publish_and_sum(s_i, t_i)

        S_tot = jnp.int32(0)
        Tp_i = jnp.int32(0)
        for j in range(NSUB):
            row = cnts_v[j, :]
            S_tot = S_tot + row[0]
            Tp_i = Tp_i + jnp.where(jnp.int32(j) < tid, row[1], 0)
        n_eq_take = jnp.int32(MAX_TOKENS) - S_tot
        quota_i = jnp.clip(n_eq_take - Tp_i, 0, t_i)
        Op_i = jnp.int32(0)
        Tp_run = jnp.int32(0)
        for j in range(NSUB):
            row = cnts_v[j, :]
            cj = row[0] + jnp.clip(n_eq_take - Tp_run, 0, row[1])
            cpj = ((cj + 7) >> 3) << 3
            Op_i = Op_i + jnp.where(jnp.int32(j) < tid, cpj, 0)
            Tp_run = Tp_run + row[1]

        def comp(i, carry):
            def active(carry):
                p, nt = carry
                k = key_v[pl.ds(i * 16, 16)]
                for l in range(16):
                    kv = k[l]
                    is_tie = kv == thr
                    sel = jnp.logical_or(
                        kv < thr, jnp.logical_and(is_tie, nt < quota_i))
                    p_eff = jnp.where(sel, p, jnp.int32(DUMP))
                    b16 = p_eff & ~jnp.int32(15)
                    ln = p_eff & jnp.int32(15)
                    old = outb_v[pl.ds(b16, 16)]
                    outb_v[pl.ds(b16, 16)] = jnp.where(
                        lanes == ln, base + i * 16 + l, old)
                    p = p + jnp.where(sel, 1, 0)
                    nt = nt + jnp.where(is_tie, 1, 0)
                return p, nt
            k = key_v[pl.ds(i * 16, 16)]
            kmin = _lane_min_u32(k, padu_v)
            return lax.cond(kmin <= thr, active, lambda c: c, carry)
        lax.fori_loop(0, NVREG, comp, (jnp.int32(0), jnp.int32(0)))

        for t in range(NSUB):
            @pl.when(tid == jnp.int32(t))
            def _():
                pltpu.sync_copy(
                    outb_v, sel_sh.at[pl.ds(pl.multiple_of(Op_i, 8), OUTPAD)])
            plsc.subcore_barrier()

        @pl.when(tid == 0)
        def _():
            Oe = jnp.int32(0)
            Op = jnp.int32(0)
            Tp2 = jnp.int32(0)
            for j in range(NSUB):
                row = cnts_v[j, :]
                cj = row[0] + jnp.clip(n_eq_take - Tp2, 0, row[1])
                pltpu.sync_copy(
                    sel_sh.at[pl.ds(pl.multiple_of(Op, 8), OUTPAD + 8)],
                    stage_v)
                def mv(q, _, cj=cj, Oe=Oe):
                    outf_v[pl.ds(Oe + q * 16, 16)] = stage_v[pl.ds(q * 16, 16)]
                    return 0
                lax.fori_loop(0, (cj + 15) >> 4, mv, 0)
                Oe = Oe + cj
                Op = Op + (((cj + 7) >> 3) << 3)
                Tp2 = Tp2 + row[1]
            pltpu.sync_copy(outf_v.at[pl.ds(0, MAX_TOKENS)], sel_hbm)

    return topk(keys)


def kernel(coordinates, features, temperature, mlp_W1, mlp_b1, mlp_W2, mlp_b2,
           imp_W1, imp_b1, ln_g, ln_b, imp_W2, imp_b2, imp_W3, imp_b3,
           nb_W1, nb_b1, nb_W2, nb_b2):
    coords4 = coordinates[:, 1:5]
    importance = _importance(
        features, coords4, mlp_W1, mlp_b1, mlp_W2, mlp_b2,
        imp_W1[:TOKEN_DIM], imp_W1[TOKEN_DIM:], imp_b1, ln_g, ln_b,
        imp_W2, imp_b2, imp_W3, imp_b3)

    sel = _sc_topk(importance)
    cents = coords4[sel]
    fsel = features[sel]
    pf_sel = jnp.maximum(fsel @ mlp_W1 + mlp_b1, 0.0) @ mlp_W2 + mlp_b2
    toks = jnp.maximum(pf_sel @ nb_W1 + nb_b1, 0.0) @ nb_W2 + nb_b2
    order = jnp.argsort(cents[:, 3])
    cents = cents[order]
    toks = toks[order]
    tokens = toks[None]
    centroids = cents[None]
    masks = jnp.ones((1, MAX_TOKENS), dtype=bool)
    return tokens, centroids, masks


# speedup vs baseline: 1.1017x; 1.0853x over previous
import functools

import jax
import jax.numpy as jnp
from jax import lax
from jax.experimental import pallas as pl
from jax.experimental.pallas import tpu as pltpu
from jax.experimental.pallas import tpu_sc as plsc

N = 262144
FEATURE_DIM = 64
TOKEN_DIM = 64
HIDDEN = 64
MAX_TOKENS = 1024
TILE = 8192


def _importance_body(feat_ref, c4_ref, mw1_ref, mb1_ref, mw2_ref, mb2_ref,
                     iw1f_ref, iw1c_ref, ib1_ref, g_ref, b_ref,
                     iw2_ref, ib2_ref, iw3_ref, ib3_ref, imp_ref):
    f = feat_ref[...]
    pf = jnp.maximum(jnp.dot(f, mw1_ref[...], preferred_element_type=jnp.float32) + mb1_ref[...], 0.0)
    pf = jnp.dot(pf, mw2_ref[...], preferred_element_type=jnp.float32) + mb2_ref[...]
    c4 = c4_ref[...]
    h = (jnp.dot(pf, iw1f_ref[...], preferred_element_type=jnp.float32)
         + jnp.dot(c4, iw1c_ref[...], preferred_element_type=jnp.float32)
         + ib1_ref[...])
    h = jnp.maximum(h, 0.0)
    mu = jnp.mean(h, axis=-1, keepdims=True)
    var = jnp.mean((h - mu) ** 2, axis=-1, keepdims=True)
    h = (h - mu) / jnp.sqrt(var + 1e-5) * g_ref[...] + b_ref[...]
    h = jnp.maximum(jnp.dot(h, iw2_ref[...], preferred_element_type=jnp.float32) + ib2_ref[...], 0.0)
    imp = jnp.dot(h, iw3_ref[...], preferred_element_type=jnp.float32) + ib3_ref[...]
    b = jax.lax.bitcast_convert_type(imp, jnp.uint32)
    neg = b >= jnp.uint32(0x80000000)
    imp_ref[...] = jnp.where(neg, b, (~b) & jnp.uint32(0x7FFFFFFF))


def _importance(features, coords4, mlp_W1, mlp_b1, mlp_W2, mlp_b2,
                imp_W1f, imp_W1c, imp_b1, ln_g, ln_b, imp_W2, imp_b2,
                imp_W3, imp_b3):
    grid = N // TILE
    wspec = lambda shape: pl.BlockSpec(shape, lambda i: tuple(0 for _ in shape))
    return pl.pallas_call(
        _importance_body,
        grid=(grid,),
        in_specs=[
            pl.BlockSpec((TILE, FEATURE_DIM), lambda i: (i, 0)),
            pl.BlockSpec((TILE, 4), lambda i: (i, 0)),
            wspec((FEATURE_DIM, HIDDEN)), wspec((HIDDEN,)),
            wspec((HIDDEN, TOKEN_DIM)), wspec((TOKEN_DIM,)),
            wspec((TOKEN_DIM, HIDDEN)), wspec((4, HIDDEN)), wspec((HIDDEN,)),
            wspec((HIDDEN,)), wspec((HIDDEN,)),
            wspec((HIDDEN, HIDDEN)), wspec((HIDDEN,)),
            wspec((HIDDEN, 1)), wspec((1,)),
        ],
        out_specs=pl.BlockSpec((TILE, 1), lambda i: (i, 0)),
        out_shape=jax.ShapeDtypeStruct((N, 1), jnp.uint32),
        compiler_params=pltpu.CompilerParams(
            dimension_semantics=("arbitrary",),
        ),
    )(features, coords4, mlp_W1, mlp_b1, mlp_W2, mlp_b2,
      imp_W1f, imp_W1c, imp_b1, ln_g, ln_b, imp_W2, imp_b2, imp_W3, imp_b3)


NSUB = 16
CHUNK = N // NSUB
NVREG = CHUNK // 16
OUTPAD = 1040
DUMP = OUTPAD - 1


def _lane_sum(v, pad_ref):
    r = v
    for s in (8, 4, 2, 1):
        pad_ref[pl.ds(0, 16)] = r
        r = r + pad_ref[pl.ds(s, 16)]
    return r[0]


def _lane_min_u32(v, pad_ref):
    r = v
    for s in (8, 4, 2, 1):
        pad_ref[pl.ds(0, 16)] = r
        r = jnp.minimum(r, pad_ref[pl.ds(s, 16)])
    return r[0]


def _sc_topk(keys):
    mesh = plsc.VectorSubcoreMesh(core_axis_name="c", subcore_axis_name="s",
                                  num_cores=1)

    @functools.partial(
        pl.kernel, mesh=mesh,
        out_type=jax.ShapeDtypeStruct((MAX_TOKENS,), jnp.int32),
        scratch_types=[
            pltpu.VMEM((CHUNK,), jnp.uint32),
            pltpu.VMEM((32,), jnp.int32),
            pltpu.VMEM((32,), jnp.uint32),
            pltpu.VMEM((16,), jnp.int32),
            pltpu.VMEM((NSUB, 16), jnp.int32),
            pltpu.VMEM((OUTPAD,), jnp.int32),
            pltpu.VMEM((OUTPAD + 8,), jnp.int32),
            pltpu.VMEM((MAX_TOKENS + 32,), jnp.int32),
            pltpu.VMEM_SHARED((NSUB, 16), jnp.int32),
            pltpu.VMEM_SHARED((2208,), jnp.int32),
        ],
    )
    def topk(key_hbm, sel_hbm, key_v, padi_v, padu_v, pub_v, cnts_v,
             outb_v, stage_v, outf_v, cnts_sh, sel_sh):
        tid = lax.axis_index("s")
        lanes = lax.iota(jnp.int32, 16)
        base = tid * CHUNK

        pltpu.sync_copy(key_hbm.at[pl.ds(base, CHUNK)], key_v)
        padi_v[pl.ds(16, 16)] = jnp.zeros((16,), jnp.int32)
        padu_v[pl.ds(16, 16)] = jnp.full((16,), 0xFFFFFFFF, jnp.uint32)

        def count_le(mid):
            def outer(i, tot):
                acc = jnp.zeros((16,), jnp.int32)
                for j in range(16):
                    k = key_v[pl.ds(i * 256 + j * 16, 16)]
                    acc = acc + jnp.where(k <= mid, 1, 0)
                return tot + _lane_sum(acc, padi_v)
            return lax.fori_loop(0, NVREG // 16, outer, jnp.int32(0))

        def publish_and_sum(val0, val1):
            pub_v[...] = jnp.where(lanes == 0, val0,
                                   jnp.where(lanes == 1, val1, 0))
            pltpu.sync_copy(pub_v, cnts_sh.at[tid])
            plsc.subcore_barrier()
            pltpu.sync_copy(cnts_sh, cnts_v)
            plsc.subcore_barrier()

        def round_(r, carry):
            lo, hi = carry
            mid = lo + lax.shift_right_logical(hi - lo, jnp.uint32(1))
            c_loc = count_le(mid)
            publish_and_sum(c_loc, jnp.int32(0))
            c_glob = jnp.int32(0)
            for j in range(NSUB):
                c_glob = c_glob + cnts_v[j, :][0]
            ge = c_glob >= jnp.int32(MAX_TOKENS)
            lo2 = jnp.where(ge, lo, mid + jnp.uint32(1))
            hi2 = jnp.where(ge, mid, hi)
            return lo2, hi2
        thr, _ = lax.fori_loop(0, 32, round_,
                               (jnp.uint32(0), jnp.uint32(0xFFFFFFFF)))

        def cnt_outer(i, carry):
            ts, tt = carry
            accs = jnp.zeros((16,), jnp.int32)
            acct = jnp.zeros((16,), jnp.int32)
            for j in range(16):
                k = key_v[pl.ds(i * 256 + j * 16, 16)]
                accs = accs + jnp.where(k < thr, 1, 0)
                acct = acct + jnp.where(k == thr, 1, 0)
            return (ts + _lane_sum(accs, padi_v), tt + _lane_sum(acct, padi_v))
        s_i, t_i = lax.fori_loop(0, NVREG // 16, cnt_outer,
                                 (jnp.int32(0), jnp.int32(0)))
        publish_and_sum(s_i, t_i)

        S_tot = jnp.int32(0)
        Tp_i = jnp.int32(0)
        for j in range(NSUB):
            row = cnts_v[j, :]
            S_tot = S_tot + row[0]
            Tp_i = Tp_i + jnp.where(jnp.int32(j) < tid, row[1], 0)
        n_eq_take = jnp.int32(MAX_TOKENS) - S_tot
        quota_i = jnp.clip(n_eq_take - Tp_i, 0, t_i)
        Op_i = jnp.int32(0)
        Tp_run = jnp.int32(0)
        for j in range(NSUB):
            row = cnts_v[j, :]
            cj = row[0] + jnp.clip(n_eq_take - Tp_run, 0, row[1])
            cpj = ((cj + 7) >> 3) << 3
            Op_i = Op_i + jnp.where(jnp.int32(j) < tid, cpj, 0)
            Tp_run = Tp_run + row[1]

        def comp(i, carry):
            def active(carry):
                p, nt = carry
                k = key_v[pl.ds(i * 16, 16)]
                for l in range(16):
                    kv = k[l]
                    is_tie = kv == thr
                    sel = jnp.logical_or(
                        kv < thr, jnp.logical_and(is_tie, nt < quota_i))
                    p_eff = jnp.where(sel, p, jnp.int32(DUMP))
                    b16 = p_eff & ~jnp.int32(15)
                    ln = p_eff & jnp.int32(15)
                    old = outb_v[pl.ds(b16, 16)]
                    outb_v[pl.ds(b16, 16)] = jnp.where(
                        lanes == ln, base + i * 16 + l, old)
                    p = p + jnp.where(sel, 1, 0)
                    nt = nt + jnp.where(is_tie, 1, 0)
                return p, nt
            k = key_v[pl.ds(i * 16, 16)]
            kmin = _lane_min_u32(k, padu_v)
            return lax.cond(kmin <= thr, active, lambda c: c, carry)
        lax.fori_loop(0, NVREG, comp, (jnp.int32(0), jnp.int32(0)))

        for t in range(NSUB):
            @pl.when(tid == jnp.int32(t))
            def _():
                pltpu.sync_copy(
                    outb_v, sel_sh.at[pl.ds(pl.multiple_of(Op_i, 8), OUTPAD)])
            plsc.subcore_barrier()

        @pl.when(tid == 0)
        def _():
            Oe = jnp.int32(0)
            Op = jnp.int32(0)
            Tp2 = jnp.int32(0)
            for j in range(NSUB):
                row = cnts_v[j, :]
                cj = row[0] + jnp.clip(n_eq_take - Tp2, 0, row[1])
                pltpu.sync_copy(
                    sel_sh.at[pl.ds(pl.multiple_of(Op, 8), OUTPAD + 8)],
                    stage_v)
                def mv(q, _, cj=cj, Oe=Oe):
                    outf_v[pl.ds(Oe + q * 16, 16)] = stage_v[pl.ds(q * 16, 16)]
                    return 0
                lax.fori_loop(0, (cj + 15) >> 4, mv, 0)
                Oe = Oe + cj
                Op = Op + (((cj + 7) >> 3) << 3)
                Tp2 = Tp2 + row[1]
            pltpu.sync_copy(outf_v.at[pl.ds(0, MAX_TOKENS)], sel_hbm)

    return topk(keys)


def kernel(coordinates, features, temperature, mlp_W1, mlp_b1, mlp_W2, mlp_b2,
           imp_W1, imp_b1, ln_g, ln_b, imp_W2, imp_b2, imp_W3, imp_b3,
           nb_W1, nb_b1, nb_W2, nb_b2):
    coords4 = coordinates[:, 1:5]
    importance = _importance(
        features, coords4, mlp_W1, mlp_b1, mlp_W2, mlp_b2,
        imp_W1[:TOKEN_DIM], imp_W1[TOKEN_DIM:], imp_b1, ln_g, ln_b,
        imp_W2, imp_b2, imp_W3, imp_b3).reshape(N)

    sel = _sc_topk(importance)
    cents = coords4[sel]
    fsel = features[sel]
    pf_sel = jnp.maximum(fsel @ mlp_W1 + mlp_b1, 0.0) @ mlp_W2 + mlp_b2
    toks = jnp.maximum(pf_sel @ nb_W1 + nb_b1, 0.0) @ nb_W2 + nb_b2
    order = jnp.argsort(cents[:, 3])
    cents = cents[order]
    toks = toks[order]
    tokens = toks[None]
    centroids = cents[None]
    masks = jnp.ones((1, MAX_TOKENS), dtype=bool)
    return tokens, centroids, masks
